# baseline (speedup 1.0000x reference)
import functools

import jax
import jax.numpy as jnp
from jax import lax
from jax.experimental import pallas as pl
from jax.experimental.pallas import tpu as pltpu
from jax.experimental.pallas import tpu_sc as plsc


def kernel(x, table):
    seq = x.shape[1]
    emb = table.shape[1]
    info = plsc.get_sparse_core_info()
    nc = info.num_cores
    rows_per_c = seq // nc
    chunk = 256
    nchunks = rows_per_c // chunk
    nbuf = 7
    mesh = plsc.ScalarSubcoreMesh(axis_name="c")

    @functools.partial(
        pl.kernel,
        out_type=jax.ShapeDtypeStruct((seq, emb), table.dtype),
        mesh=mesh,
        scratch_types=[
            pltpu.VMEM_SHARED((nbuf, chunk, emb), jnp.float32),
            pltpu.SemaphoreType.DMA,
            pltpu.SemaphoreType.DMA,
        ],
    )
    def sc_copy(table_hbm, out_hbm, buf, in_sem, out_sem):
        cid = lax.axis_index("c")
        base = cid * rows_per_c

        def in_copy(i, slot):
            return pltpu.make_async_copy(
                table_hbm.at[pl.ds(base + i * chunk, chunk)], buf.at[slot], in_sem
            )

        def out_copy(i, slot):
            return pltpu.make_async_copy(
                buf.at[slot], out_hbm.at[pl.ds(base + i * chunk, chunk)], out_sem
            )

        out_waited = [False] * nchunks
        for i in range(min(nbuf, nchunks)):
            in_copy(i, i).start()
        for i in range(nchunks):
            slot = i % nbuf
            in_copy(i, slot).wait()
            out_copy(i, slot).start()
            nxt = i + nbuf
            if nxt < nchunks:
                out_copy(i, slot).wait()
                out_waited[i] = True
                in_copy(nxt, slot).start()
        for i in range(nchunks):
            if not out_waited[i]:
                out_copy(i, i % nbuf).wait()

    return sc_copy(table)[None, :, :]
